# baseline (speedup 1.0000x reference)
import functools

import jax
import jax.numpy as jnp
from jax import lax
from jax.experimental import pallas as pl
from jax.experimental.pallas import tpu as pltpu
from jax.experimental.pallas import tpu_sc as plsc

_F = 64
_C = 100000
_B = 1024
_BETA = 0.05
_K = 6
_C_BLK = 2048
_NBLK = -(-_C // _C_BLK)
_NCHUNK = _C_BLK // 128
_NEG = -1e30
_LOG2E = 1.4426950408889634
_LN2 = 0.6931471805599453


def _sc_gather(em_pairs, idx):
    info = plsc.get_sparse_core_info()
    nc, ns = info.num_cores, info.num_subcores
    nw = nc * ns
    bpw = _B // nw
    mesh = plsc.VectorSubcoreMesh(core_axis_name="c", subcore_axis_name="s")

    @functools.partial(
        pl.kernel,
        mesh=mesh,
        out_type=jax.ShapeDtypeStruct((_B, 2 * _F), jnp.float32),
        scratch_types=[
            pltpu.VMEM((bpw,), jnp.int32),
            pltpu.VMEM((bpw, 2 * _F), jnp.float32),
            pltpu.SemaphoreType.DMA,
        ],
    )
    def gather_kernel(table_hbm, idx_hbm, out_hbm, idx_v, rows_v, sem):
        wid = lax.axis_index("s") * nc + lax.axis_index("c")
        base = wid * bpw
        pltpu.sync_copy(idx_hbm.at[pl.ds(base, bpw)], idx_v)
        pltpu.async_copy(table_hbm.at[idx_v], rows_v, sem).wait()
        pltpu.sync_copy(rows_v, out_hbm.at[pl.ds(base, bpw)])

    return gather_kernel(em_pairs, idx)


def _top6_rounds(x):
    tops = []
    for k in range(_K):
        mk = jnp.max(x, axis=1, keepdims=True)
        tops.append(mk)
        if k + 1 < _K:
            x = jnp.where(x == mk, _NEG, x)
    return jnp.concatenate(tops, axis=1)


def _merge_top(r, s, keep):
    nr, ns = len(r), len(s)
    out = []
    for i in range(keep):
        cands = []
        for j in range(-1, nr):
            k = i - 1 - j
            if k < -1 or k >= ns:
                continue
            if j == -1:
                cands.append(s[k])
            elif k == -1:
                cands.append(r[j])
            else:
                cands.append(jnp.minimum(r[j], s[k]))
        m = cands[0]
        for c in cands[1:]:
            m = jnp.maximum(m, c)
        out.append(m)
    return out


def _oem_merge(a, b):
    n = len(a)
    if n == 1:
        return [jnp.maximum(a[0], b[0]), jnp.minimum(a[0], b[0])]
    ev = _oem_merge(a[0::2], b[0::2])
    od = _oem_merge(a[1::2], b[1::2])
    out = [ev[0]]
    for i in range(1, n):
        out.append(jnp.maximum(od[i - 1], ev[i]))
        out.append(jnp.minimum(od[i - 1], ev[i]))
    out.append(od[n - 1])
    return out


def _block_slot_top6(logits):
    chunks = [logits[:, c * 128:(c + 1) * 128] for c in range(_NCHUNK)]
    lists = [[c] for c in chunks]
    while len(lists) > 2 and len(lists[0]) < 8:
        lists = [_oem_merge(lists[j], lists[j + 1])
                 for j in range(0, len(lists), 2)]
    lists = [l[:_K] for l in lists]
    while len(lists) > 1:
        lists = [_merge_top(lists[j], lists[j + 1], _K)
                 for j in range(0, len(lists), 2)]
    return lists[0]


def _tc_body(x_ref, em_ref, out_ref, m_ref, s_ref, t6_ref):
    i = pl.program_id(0)

    @pl.when(i == 0)
    def _init():
        m_ref[...] = jnp.full((_B, 1), _NEG, jnp.float32)
        s_ref[...] = jnp.zeros((_B, 1), jnp.float32)
        t6_ref[...] = jnp.full((_K, _B, 128), _NEG, jnp.float32)

    a = x_ref[...] * (_LOG2E / _BETA)
    e = em_ref[...]
    row = lax.broadcasted_iota(jnp.int32, (_C_BLK, _F), 0) + i * _C_BLK
    e = jnp.where(row < _C, e, 0.0)
    logits = lax.dot_general(a, e, (((1,), (1,)), ((), ())),
                             preferred_element_type=jnp.float32)

    blk6 = _block_slot_top6(logits)
    run = [t6_ref[k] for k in range(_K)]
    new_run = _merge_top(run, blk6, _K)
    for k in range(_K):
        t6_ref[k] = new_run[k]

    m_old = m_ref[...]
    m_new = jnp.max(new_run[0], axis=1, keepdims=True)
    s_ref[...] = s_ref[...] * jnp.exp2(m_old - m_new) + jnp.sum(
        jnp.exp2(logits - m_new), axis=1, keepdims=True)
    m_ref[...] = m_new

    @pl.when(i == _NBLK - 1)
    def _fin():
        lse = m_ref[...] + jnp.log(s_ref[...]) * _LOG2E
        cand = jnp.concatenate([t6_ref[k] for k in range(_K)], axis=1)
        t6 = _top6_rounds(cand)
        v6 = t6[:, _K - 1:_K]
        sum6 = jnp.sum(t6, axis=1, keepdims=True)
        out_ref[...] = jnp.concatenate([lse, v6, sum6], axis=1)


def _fin_body(x_ref, g_ref, t_ref, st_ref, out_ref):
    a = x_ref[...] * (_LOG2E / _BETA)
    par = t_ref[...] % 2
    lane = lax.broadcasted_iota(jnp.int32, (_B, 2 * _F), 1)
    sel = (lane < _F) == (par == 0)
    a2 = jnp.concatenate([a, a], axis=1)
    lt = jnp.sum(jnp.where(sel, g_ref[...] * a2, 0.0),
                 axis=1, keepdims=True)
    lse = st_ref[:, 0:1]
    v6 = st_ref[:, 1:2]
    sum6 = st_ref[:, 2:3]
    in_top = lt >= v6
    sum_wo = sum6 - jnp.where(in_top, lt, 0.0)
    cnt = jnp.where(in_top, float(_K - 1), float(_K))
    loss_row = (1.0 + cnt / _K) * lse - lt - sum_wo / _K
    out_ref[...] = jnp.sum(loss_row, axis=0, keepdims=True) * (_LN2 / _B)


def _tc_call(inputs, em, interpret=False):
    return pl.pallas_call(
        _tc_body,
        grid=(_NBLK,),
        in_specs=[
            pl.BlockSpec((_B, _F), lambda i: (0, 0)),
            pl.BlockSpec((_C_BLK, _F), lambda i: (i, 0)),
        ],
        out_specs=pl.BlockSpec((_B, 3), lambda i: (0, 0)),
        out_shape=jax.ShapeDtypeStruct((_B, 3), jnp.float32),
        scratch_shapes=[
            pltpu.VMEM((_B, 1), jnp.float32),
            pltpu.VMEM((_B, 1), jnp.float32),
            pltpu.VMEM((_K, _B, 128), jnp.float32),
        ],
        compiler_params=pltpu.CompilerParams(
            dimension_semantics=("arbitrary",),
        ),
        interpret=interpret,
    )(inputs, em)


def _fin_call(inputs, gathered, targets2d, stats, interpret=False):
    return pl.pallas_call(
        _fin_body,
        out_shape=jax.ShapeDtypeStruct((1, 1), jnp.float32),
        interpret=interpret,
    )(inputs, gathered, targets2d, stats)


def kernel(inputs, em, targets, epoch):
    em_pairs = em.reshape(_C // 2, 2 * _F)
    gathered = _sc_gather(em_pairs, targets >> 1)
    stats = _tc_call(inputs, em)
    out = _fin_call(inputs, gathered, targets.reshape(_B, 1), stats)
    return out[0, 0]


# speedup vs baseline: 1.0070x; 1.0070x over previous
import functools

import jax
import jax.numpy as jnp
from jax import lax
from jax.experimental import pallas as pl
from jax.experimental.pallas import tpu as pltpu
from jax.experimental.pallas import tpu_sc as plsc

_F = 64
_C = 100000
_B = 1024
_BETA = 0.05
_K = 6
_C_BLK = 2048
_NBLK = -(-_C // _C_BLK)
_NCHUNK = _C_BLK // 128
_NEG = -1e30
_LOG2E = 1.4426950408889634
_LN2 = 0.6931471805599453


def _sc_gather(em_pairs, idx):
    info = plsc.get_sparse_core_info()
    nc, ns = info.num_cores, info.num_subcores
    nw = nc * ns
    bpw = _B // nw
    mesh = plsc.VectorSubcoreMesh(core_axis_name="c", subcore_axis_name="s")

    @functools.partial(
        pl.kernel,
        mesh=mesh,
        out_type=jax.ShapeDtypeStruct((_B, 2 * _F), jnp.float32),
        scratch_types=[
            pltpu.VMEM((bpw,), jnp.int32),
            pltpu.VMEM((bpw, 2 * _F), jnp.float32),
            pltpu.SemaphoreType.DMA,
        ],
    )
    def gather_kernel(table_hbm, idx_hbm, out_hbm, idx_v, rows_v, sem):
        wid = lax.axis_index("s") * nc + lax.axis_index("c")
        base = wid * bpw
        pltpu.sync_copy(idx_hbm.at[pl.ds(base, bpw)], idx_v)
        pltpu.async_copy(table_hbm.at[idx_v], rows_v, sem).wait()
        pltpu.sync_copy(rows_v, out_hbm.at[pl.ds(base, bpw)])

    return gather_kernel(em_pairs, idx)


def _top6_rounds(x):
    tops = []
    for k in range(_K):
        mk = jnp.max(x, axis=1, keepdims=True)
        tops.append(mk)
        if k + 1 < _K:
            x = jnp.where(x == mk, _NEG, x)
    return jnp.concatenate(tops, axis=1)


def _merge_top(r, s, keep):
    nr, ns = len(r), len(s)
    out = []
    for i in range(keep):
        cands = []
        for j in range(-1, nr):
            k = i - 1 - j
            if k < -1 or k >= ns:
                continue
            if j == -1:
                cands.append(s[k])
            elif k == -1:
                cands.append(r[j])
            else:
                cands.append(jnp.minimum(r[j], s[k]))
        m = cands[0]
        for c in cands[1:]:
            m = jnp.maximum(m, c)
        out.append(m)
    return out


def _oem_merge(a, b):
    n = len(a)
    if n == 1:
        return [jnp.maximum(a[0], b[0]), jnp.minimum(a[0], b[0])]
    ev = _oem_merge(a[0::2], b[0::2])
    od = _oem_merge(a[1::2], b[1::2])
    out = [ev[0]]
    for i in range(1, n):
        out.append(jnp.maximum(od[i - 1], ev[i]))
        out.append(jnp.minimum(od[i - 1], ev[i]))
    out.append(od[n - 1])
    return out


def _block_slot_top6(logits):
    chunks = [logits[:, c * 128:(c + 1) * 128] for c in range(_NCHUNK)]
    lists = [[c] for c in chunks]
    while len(lists) > 2 and len(lists[0]) < 8:
        lists = [_oem_merge(lists[j], lists[j + 1])
                 for j in range(0, len(lists), 2)]
    lists = [l[:_K] for l in lists]
    while len(lists) > 1:
        lists = [_merge_top(lists[j], lists[j + 1], _K)
                 for j in range(0, len(lists), 2)]
    return lists[0]


def _tc_body(x_ref, g_ref, t_ref, em_ref, out_ref, m_ref, s_ref, t6_ref):
    i = pl.program_id(0)

    @pl.when(i == 0)
    def _init():
        m_ref[...] = jnp.full((_B, 1), _NEG, jnp.float32)
        s_ref[...] = jnp.zeros((_B, 1), jnp.float32)
        t6_ref[...] = jnp.full((_K, _B, 128), _NEG, jnp.float32)

    a = x_ref[...] * (_LOG2E / _BETA)
    e = em_ref[...]
    row = lax.broadcasted_iota(jnp.int32, (_C_BLK, _F), 0) + i * _C_BLK
    e = jnp.where(row < _C, e, 0.0)
    logits = lax.dot_general(a, e, (((1,), (1,)), ((), ())),
                             preferred_element_type=jnp.float32)

    blk6 = _block_slot_top6(logits)
    run = [t6_ref[k] for k in range(_K)]
    new_run = _merge_top(run, blk6, _K)
    for k in range(_K):
        t6_ref[k] = new_run[k]

    m_old = m_ref[...]
    m_new = jnp.max(new_run[0], axis=1, keepdims=True)
    s_ref[...] = s_ref[...] * jnp.exp2(m_old - m_new) + jnp.sum(
        jnp.exp2(logits - m_new), axis=1, keepdims=True)
    m_ref[...] = m_new

    @pl.when(i == _NBLK - 1)
    def _fin():
        lse = m_ref[...] + jnp.log(s_ref[...]) * _LOG2E
        par = t_ref[...] % 2
        lane = lax.broadcasted_iota(jnp.int32, (_B, 2 * _F), 1)
        sel = (lane < _F) == (par == 0)
        a2 = jnp.concatenate([a, a], axis=1)
        lt = jnp.sum(jnp.where(sel, g_ref[...] * a2, 0.0),
                     axis=1, keepdims=True)
        cand = jnp.concatenate([t6_ref[k] for k in range(_K)], axis=1)
        t6 = _top6_rounds(cand)
        v6 = t6[:, _K - 1:_K]
        sum6 = jnp.sum(t6, axis=1, keepdims=True)
        in_top = lt >= v6
        sum_wo = sum6 - jnp.where(in_top, lt, 0.0)
        cnt = jnp.where(in_top, float(_K - 1), float(_K))
        loss_row = (1.0 + cnt / _K) * lse - lt - sum_wo / _K
        out_ref[...] = jnp.sum(loss_row, axis=0, keepdims=True) * (_LN2 / _B)


def _tc_call(inputs, gathered, targets2d, em, interpret=False):
    return pl.pallas_call(
        _tc_body,
        grid=(_NBLK,),
        in_specs=[
            pl.BlockSpec((_B, _F), lambda i: (0, 0)),
            pl.BlockSpec((_B, 2 * _F), lambda i: (0, 0)),
            pl.BlockSpec((_B, 1), lambda i: (0, 0)),
            pl.BlockSpec((_C_BLK, _F), lambda i: (i, 0)),
        ],
        out_specs=pl.BlockSpec((1, 1), lambda i: (0, 0)),
        out_shape=jax.ShapeDtypeStruct((1, 1), jnp.float32),
        scratch_shapes=[
            pltpu.VMEM((_B, 1), jnp.float32),
            pltpu.VMEM((_B, 1), jnp.float32),
            pltpu.VMEM((_K, _B, 128), jnp.float32),
        ],
        compiler_params=pltpu.CompilerParams(
            dimension_semantics=("arbitrary",),
        ),
        interpret=interpret,
    )(inputs, gathered, targets2d, em)


def kernel(inputs, em, targets, epoch):
    em_pairs = em.reshape(_C // 2, 2 * _F)
    gathered = _sc_gather(em_pairs, targets >> 1)
    out = _tc_call(inputs, gathered, targets.reshape(_B, 1), em)
    return out[0, 0]
